# Initial kernel scaffold; baseline (speedup 1.0000x reference)
import functools

import jax
import jax.numpy as jnp
from jax import lax
from jax.experimental import pallas as pl
from jax.experimental.pallas import tpu as pltpu
from jax.experimental.pallas import tpu_sc as plsc

B = 16384
F = 26
V = 100000
D = 64
BF = B * F

_info = plsc.get_sparse_core_info()
NC, NS, L = _info.num_cores, _info.num_subcores, _info.num_lanes
NW = NC * NS

CB = 16
CH = CB * F
G = 104
NG = CH // G
B_PER_W = B // NW
NCHUNK = B_PER_W // CB
ROWS_PER_W = B_PER_W * F


def _sc_embedding(sparse_hbm, tables_hbm, fixed_hbm, out_hbm,
                  raw_v, idx_v, rows_v, offs_v, fids_v, acc_v, accg_v,
                  fv_v, sem):
    wid = lax.axis_index("s") * NC + lax.axis_index("c")
    lanes = lax.iota(jnp.int32, L)

    for i in range(CH // L):
        f = jnp.remainder(i * L + lanes, F)
        fids_v[pl.ds(i * L, L)] = f
        offs_v[pl.ds(i * L, L)] = f * V

    acc_v[pl.ds(0, L)] = jnp.zeros((L,), jnp.int32)
    acc_v[pl.ds(L, L)] = jnp.where(lanes < F - L, 0, 1)

    pltpu.sync_copy(fixed_hbm, fv_v)

    def chunk_body(c, carry):
        base = (wid * NCHUNK + c) * CH
        pltpu.sync_copy(sparse_hbm.at[pl.ds(base, CH)], raw_v)
        for i in range(CH // L):
            sl = pl.ds(i * L, L)
            vals = raw_v[sl]
            idx_v[sl] = vals + offs_v[sl]
            plsc.addupdate_scatter(acc_v, [fids_v[sl]], vals)
        cps = [
            pltpu.async_copy(
                tables_hbm.at[idx_v.at[pl.ds(g * G, G)]],
                rows_v.at[pl.ds(g * G, G)], sem)
            for g in range(NG)
        ]
        for cp in cps:
            cp.wait()
        pltpu.sync_copy(rows_v, out_hbm.at[pl.ds(base, CH)])
        return carry

    lax.fori_loop(0, NCHUNK, chunk_body, 0)

    a0 = acc_v[pl.ds(0, L)]
    a1 = acc_v[pl.ds(L, L)]
    nzero = (jnp.sum(jnp.where(a0 == 0, 1, 0))
             + jnp.sum(jnp.where(a1 == 0, 1, 0)))

    @pl.when(nzero > 0)
    def _rare():
        accg_v[pl.ds(0, L)] = jnp.zeros((L,), jnp.int32)
        accg_v[pl.ds(L, L)] = jnp.where(lanes < F - L, 0, 1)

        def scan_body(c, carry):
            pltpu.sync_copy(sparse_hbm.at[pl.ds(c * CH, CH)], raw_v)
            for i in range(CH // L):
                sl = pl.ds(i * L, L)
                plsc.addupdate_scatter(accg_v, [fids_v[sl]], raw_v[sl])
            return carry

        lax.fori_loop(0, BF // CH, scan_body, 0)

        g0 = accg_v[pl.ds(0, L)]
        g1 = accg_v[pl.ds(L, L)]
        for f in range(F):
            vec = g0 if f < L else g1
            sel = jnp.where(lanes == (f % L), vec, 1)
            colsum_like = jnp.min(sel)

            @pl.when(colsum_like == 0)
            def _overwrite():
                def row_body(b, carry):
                    row = wid * ROWS_PER_W + b * F + f
                    pltpu.sync_copy(fv_v, out_hbm.at[pl.ds(row, 1)])
                    return carry
                lax.fori_loop(0, B_PER_W, row_body, 0)


@jax.jit
def kernel(sparse_inputs, tables, fixed_vector):
    sparse_flat = sparse_inputs.reshape(BF)
    tables_flat = tables.reshape(F * V, D)
    fixed2 = fixed_vector.reshape(1, D)

    run = functools.partial(
        pl.kernel,
        mesh=plsc.VectorSubcoreMesh(core_axis_name="c", subcore_axis_name="s"),
        out_type=jax.ShapeDtypeStruct((BF, D), jnp.float32),
        scratch_types=[
            pltpu.VMEM((CH,), jnp.int32),
            pltpu.VMEM((CH,), jnp.int32),
            pltpu.VMEM((CH, D), jnp.float32),
            pltpu.VMEM((CH,), jnp.int32),
            pltpu.VMEM((CH,), jnp.int32),
            pltpu.VMEM((2 * L,), jnp.int32),
            pltpu.VMEM((2 * L,), jnp.int32),
            pltpu.VMEM((1, D), jnp.float32),
            pltpu.SemaphoreType.DMA,
        ],
    )(_sc_embedding)

    out_flat = run(sparse_flat, tables_flat, fixed2)
    return out_flat.reshape(B, F, D)


# speedup vs baseline: 1.0019x; 1.0019x over previous
import functools

import jax
import jax.numpy as jnp
from jax import lax
from jax.experimental import pallas as pl
from jax.experimental.pallas import tpu as pltpu
from jax.experimental.pallas import tpu_sc as plsc

B = 16384
F = 26
V = 100000
D = 64
BF = B * F

_info = plsc.get_sparse_core_info()
NC, NS, L = _info.num_cores, _info.num_subcores, _info.num_lanes
NW = NC * NS

CB = 16
CH = CB * F
G = 104
NG = CH // G
B_PER_W = B // NW
NCHUNK = B_PER_W // CB
ROWS_PER_W = B_PER_W * F
NBIN = 208
NBV = NBIN // L


def _sc_embedding(sparse_hbm, tables_hbm, fixed_hbm, out_hbm,
                  raw_v, idx_v, rows_v, offs_v, acc_v, fv_v, sem):
    wid = lax.axis_index("s") * NC + lax.axis_index("c")
    lanes = lax.iota(jnp.int32, L)
    zero_l = jnp.zeros((L,), jnp.int32)

    for i in range(CH // L):
        offs_v[pl.ds(i * L, L)] = jnp.remainder(i * L + lanes, F) * V
    for i in range(NBV):
        acc_v[pl.ds(i * L, L)] = zero_l

    pltpu.sync_copy(fixed_hbm, fv_v)

    def chunk_body(c, carry):
        base = (wid * NCHUNK + c) * CH
        pltpu.sync_copy(sparse_hbm.at[pl.ds(base, CH)], raw_v)
        for i in range(CH // L):
            sl = pl.ds(i * L, L)
            vals = raw_v[sl]
            idx_v[sl] = vals + offs_v[sl]
            asl = pl.ds((i % NBV) * L, L)
            acc_v[asl] = acc_v[asl] + vals
        cps = [
            pltpu.async_copy(
                tables_hbm.at[idx_v.at[pl.ds(g * G, G)]],
                rows_v.at[pl.ds(g * G, G)], sem)
            for g in range(NG)
        ]
        for cp in cps:
            cp.wait()
        pltpu.sync_copy(rows_v, out_hbm.at[pl.ds(base, CH)])
        return carry

    lax.fori_loop(0, NCHUNK, chunk_body, 0)

    zc = jnp.where(acc_v[pl.ds(0, L)] == 0, 1, 0)
    for i in range(1, NBV):
        zc = zc | jnp.where(acc_v[pl.ds(i * L, L)] == 0, 1, 0)
    any_zero = zc[0]
    for l in range(1, L):
        any_zero = any_zero | zc[l]

    @pl.when(any_zero > 0)
    def _rare():
        for i in range(NBV):
            acc_v[pl.ds(i * L, L)] = zero_l

        def scan_body(c, carry):
            pltpu.sync_copy(sparse_hbm.at[pl.ds(c * CH, CH)], raw_v)
            for i in range(CH // L):
                asl = pl.ds((i % NBV) * L, L)
                acc_v[asl] = acc_v[asl] + raw_v[pl.ds(i * L, L)]
            return carry

        lax.fori_loop(0, BF // CH, scan_body, 0)

        for f in range(F):
            colsum = jnp.int32(0)
            for k in range(NBIN // F):
                p = f + k * F
                vec = acc_v[pl.ds((p // L) * L, L)]
                colsum = colsum + vec[p % L]

            @pl.when(colsum == 0)
            def _overwrite():
                def row_body(b, carry):
                    row = wid * ROWS_PER_W + b * F + f
                    pltpu.sync_copy(fv_v, out_hbm.at[pl.ds(row, 1)])
                    return carry
                lax.fori_loop(0, B_PER_W, row_body, 0)


@jax.jit
def kernel(sparse_inputs, tables, fixed_vector):
    sparse_flat = sparse_inputs.reshape(BF)
    tables_flat = tables.reshape(F * V, D)
    fixed2 = fixed_vector.reshape(1, D)

    run = functools.partial(
        pl.kernel,
        mesh=plsc.VectorSubcoreMesh(core_axis_name="c", subcore_axis_name="s"),
        out_type=jax.ShapeDtypeStruct((BF, D), jnp.float32),
        compiler_params=pltpu.CompilerParams(use_tc_tiling_on_sc=False),
        scratch_types=[
            pltpu.VMEM((CH,), jnp.int32),
            pltpu.VMEM((CH,), jnp.int32),
            pltpu.VMEM((CH, D), jnp.float32),
            pltpu.VMEM((CH,), jnp.int32),
            pltpu.VMEM((NBIN,), jnp.int32),
            pltpu.VMEM((1, D), jnp.float32),
            pltpu.SemaphoreType.DMA,
        ],
    )(_sc_embedding)

    out_flat = run(sparse_flat, tables_flat, fixed2)
    return out_flat.reshape(B, F, D)


# speedup vs baseline: 3.5359x; 3.5290x over previous
import functools

import jax
import jax.numpy as jnp
from jax import lax
from jax.experimental import pallas as pl
from jax.experimental.pallas import tpu as pltpu
from jax.experimental.pallas import tpu_sc as plsc

B = 16384
F = 26
V = 100000
D = 64

_info = plsc.get_sparse_core_info()
NC, NS, L = _info.num_cores, _info.num_subcores, _info.num_lanes
NW = NC * NS
UNITS = F * D
UPW = UNITS // NW
BH = B // 2


def _sc_embedding(spT_hbm, tt_hbm, out_hbm, src_v, idx_v, out_v, sem):
    wid = lax.axis_index("s") * NC + lax.axis_index("c")

    def unit_body(j, carry):
        prev_f, flag = carry
        u = wid * UPW + j
        f = u // D
        d = u % D

        @pl.when(f != prev_f)
        def _load_idx():
            pltpu.sync_copy(spT_hbm.at[f, :], idx_v)

        def new_flag():
            def acc_body(k, acc):
                return acc + idx_v[pl.ds(k * L, L)]
            acc = lax.fori_loop(0, B // L, acc_body,
                                jnp.zeros((L,), jnp.int32))
            s = acc[0]
            for l in range(1, L):
                s = s + acc[l]
            return (s == 0).astype(jnp.int32)

        flag = lax.cond(f != prev_f, new_flag, lambda: flag)

        pltpu.sync_copy(tt_hbm.at[f, d, :], src_v)

        @pl.when(flag == 0)
        def _gather():
            for half in range(2):
                def g_body(k, carry2):
                    base = k * (8 * L)
                    for t in range(8):
                        sl = pl.ds(base + t * L, L)
                        iv = idx_v[pl.ds(half * BH + base + t * L, L)]
                        out_v[sl] = plsc.load_gather(src_v, [iv])
                    return carry2
                lax.fori_loop(0, BH // (8 * L), g_body, 0)
                pltpu.sync_copy(out_v, out_hbm.at[f, d, pl.ds(half * BH, BH)])

        @pl.when(flag == 1)
        def _ones():
            ones_l = jnp.ones((L,), jnp.float32)

            def o_body(k, carry2):
                base = k * (8 * L)
                for t in range(8):
                    out_v[pl.ds(base + t * L, L)] = ones_l
                return carry2
            lax.fori_loop(0, BH // (8 * L), o_body, 0)
            for half in range(2):
                pltpu.sync_copy(out_v, out_hbm.at[f, d, pl.ds(half * BH, BH)])

        return (f, flag)

    lax.fori_loop(0, UPW, unit_body, (jnp.int32(-1), jnp.int32(0)))


@jax.jit
def kernel(sparse_inputs, tables, fixed_vector):
    del fixed_vector
    spT = sparse_inputs.T
    tt = jnp.transpose(tables, (0, 2, 1))

    run = functools.partial(
        pl.kernel,
        mesh=plsc.VectorSubcoreMesh(core_axis_name="c", subcore_axis_name="s"),
        out_type=jax.ShapeDtypeStruct((F, D, B), jnp.float32),
        compiler_params=pltpu.CompilerParams(use_tc_tiling_on_sc=True,
                                             needs_layout_passes=False),
        scratch_types=[
            pltpu.VMEM((V,), jnp.float32),
            pltpu.VMEM((B,), jnp.int32),
            pltpu.VMEM((BH,), jnp.float32),
            pltpu.SemaphoreType.DMA,
        ],
    )(_sc_embedding)

    outp = run(spT, tt)
    return jnp.transpose(outp, (2, 0, 1))


# speedup vs baseline: 3.7689x; 1.0659x over previous
import functools

import jax
import jax.numpy as jnp
from jax import lax
from jax.experimental import pallas as pl
from jax.experimental.pallas import tpu as pltpu
from jax.experimental.pallas import tpu_sc as plsc

B = 16384
F = 26
V = 100000
D = 64

_info = plsc.get_sparse_core_info()
NC, NS, L = _info.num_cores, _info.num_subcores, _info.num_lanes
NW = NC * NS
UNITS = F * D
UPW = UNITS // NW
NQ = 4
BQ = B // NQ


def _sc_embedding(spT_hbm, tt_hbm, out_hbm, src_v, idx_v, outA_v, outB_v,
                  semA, semB):
    wid = lax.axis_index("s") * NC + lax.axis_index("c")
    obufs = (outA_v, semA), (outB_v, semB)

    def unit_body(j, carry):
        prev_f, flag = carry
        u = wid * UPW + j
        f = u // D
        d = u % D

        @pl.when(f != prev_f)
        def _load_idx():
            pltpu.sync_copy(spT_hbm.at[f, :], idx_v)

        def new_flag():
            def acc_body(k, acc):
                return acc + idx_v[pl.ds(k * L, L)]
            acc = lax.fori_loop(0, B // L, acc_body,
                                jnp.zeros((L,), jnp.int32))
            s = acc[0]
            for l in range(1, L):
                s = s + acc[l]
            return (s == 0).astype(jnp.int32)

        flag = lax.cond(f != prev_f, new_flag, lambda: flag)

        pltpu.sync_copy(tt_hbm.at[f, d, :], src_v)

        for q in range(NQ):
            ov, sm = obufs[q % 2]

            def _drain():
                pltpu.make_async_copy(
                    out_hbm.at[0, 0, pl.ds(0, BQ)], ov, sm).wait()

            if q >= 2:
                _drain()
            else:
                pl.when(j > 0)(_drain)

            @pl.when(flag == 0)
            def _gather():
                def g_body(k, carry2):
                    base = k * (8 * L)
                    for t in range(8):
                        sl = pl.ds(base + t * L, L)
                        iv = idx_v[pl.ds(q * BQ + base + t * L, L)]
                        ov[sl] = plsc.load_gather(src_v, [iv])
                    return carry2
                lax.fori_loop(0, BQ // (8 * L), g_body, 0)

            @pl.when(flag == 1)
            def _ones():
                ones_l = jnp.ones((L,), jnp.float32)

                def o_body(k, carry2):
                    base = k * (8 * L)
                    for t in range(8):
                        ov[pl.ds(base + t * L, L)] = ones_l
                    return carry2
                lax.fori_loop(0, BQ // (8 * L), o_body, 0)

            pltpu.async_copy(ov, out_hbm.at[f, d, pl.ds(q * BQ, BQ)], sm)

        return (f, flag)

    lax.fori_loop(0, UPW, unit_body, (jnp.int32(-1), jnp.int32(0)))
    for ov, sm in obufs:
        pltpu.make_async_copy(out_hbm.at[0, 0, pl.ds(0, BQ)], ov, sm).wait()


@jax.jit
def kernel(sparse_inputs, tables, fixed_vector):
    del fixed_vector
    spT = sparse_inputs.T
    tt = jnp.transpose(tables, (0, 2, 1))

    run = functools.partial(
        pl.kernel,
        mesh=plsc.VectorSubcoreMesh(core_axis_name="c", subcore_axis_name="s"),
        out_type=jax.ShapeDtypeStruct((F, D, B), jnp.float32),
        compiler_params=pltpu.CompilerParams(use_tc_tiling_on_sc=True,
                                             needs_layout_passes=False),
        scratch_types=[
            pltpu.VMEM((V,), jnp.float32),
            pltpu.VMEM((B,), jnp.int32),
            pltpu.VMEM((BQ,), jnp.float32),
            pltpu.VMEM((BQ,), jnp.float32),
            pltpu.SemaphoreType.DMA,
            pltpu.SemaphoreType.DMA,
        ],
    )(_sc_embedding)

    outp = run(spT, tt)
    return jnp.transpose(outp, (2, 0, 1))
